# baseline (speedup 1.0000x reference)
import jax
import jax.numpy as jnp
from jax.experimental import pallas as pl
from jax.experimental.pallas import tpu as pltpu


def _fused_body(x_ref, w_ref, adj_ref, bias_ref, o_ref, s_ref):
    i = pl.program_id(0)

    @pl.when(i == 0)
    def _compute_support():
        w = w_ref[...]
        for b in range(x_ref.shape[0]):
            s_ref[b] = jnp.dot(x_ref[b], w, preferred_element_type=jnp.float32)

    a = adj_ref[...]
    bias = bias_ref[...]
    for b in range(s_ref.shape[0]):
        o_ref[b] = jnp.dot(a, s_ref[b],
                           preferred_element_type=jnp.float32) + bias


def kernel(batch_input, adj, weight, bias):
    if batch_input.ndim == 2:
        batch_input = batch_input[None]
    nb, n, f_in = batch_input.shape
    f_out = weight.shape[1]
    m = adj.shape[0]

    bm = 200 if m % 200 == 0 else m
    bias2d = bias.reshape(1, f_out)

    out = pl.pallas_call(
        _fused_body,
        grid=(m // bm,),
        in_specs=[
            pl.BlockSpec((nb, n, f_in), lambda i: (0, 0, 0)),
            pl.BlockSpec((f_in, f_out), lambda i: (0, 0)),
            pl.BlockSpec((bm, n), lambda i: (i, 0)),
            pl.BlockSpec((1, f_out), lambda i: (0, 0)),
        ],
        out_specs=pl.BlockSpec((nb, bm, f_out), lambda i: (0, i, 0)),
        out_shape=jax.ShapeDtypeStruct((nb, m, f_out), jnp.float32),
        scratch_shapes=[pltpu.VMEM((nb, n, f_out), jnp.float32)],
        compiler_params=pltpu.CompilerParams(
            dimension_semantics=("arbitrary",),
        ),
    )(batch_input, weight, adj, bias2d)

    return out


# speedup vs baseline: 1.1912x; 1.1912x over previous
import jax
import jax.numpy as jnp
from jax.experimental import pallas as pl
from jax.experimental.pallas import tpu as pltpu


def _fused_body(x_ref, w_ref, adj_ref, bias_ref, o_ref, s_ref):
    i = pl.program_id(0)
    nb = x_ref.shape[0]
    f_out = w_ref.shape[1]

    @pl.when(i == 0)
    def _compute_support():
        w = w_ref[...]
        for b in range(nb):
            s_ref[:, b * f_out:(b + 1) * f_out] = jnp.dot(
                x_ref[b], w, preferred_element_type=jnp.float32)

    a = adj_ref[...]
    bias = bias_ref[...]
    o_cat = jnp.dot(a, s_ref[...], preferred_element_type=jnp.float32)
    for b in range(nb):
        o_ref[b] = o_cat[:, b * f_out:(b + 1) * f_out] + bias


def kernel(batch_input, adj, weight, bias):
    if batch_input.ndim == 2:
        batch_input = batch_input[None]
    nb, n, f_in = batch_input.shape
    f_out = weight.shape[1]
    m = adj.shape[0]

    bm = 400 if m % 400 == 0 else m
    bias2d = bias.reshape(1, f_out)

    out = pl.pallas_call(
        _fused_body,
        grid=(m // bm,),
        in_specs=[
            pl.BlockSpec((nb, n, f_in), lambda i: (0, 0, 0)),
            pl.BlockSpec((f_in, f_out), lambda i: (0, 0)),
            pl.BlockSpec((bm, n), lambda i: (i, 0)),
            pl.BlockSpec((1, f_out), lambda i: (0, 0)),
        ],
        out_specs=pl.BlockSpec((nb, bm, f_out), lambda i: (0, i, 0)),
        out_shape=jax.ShapeDtypeStruct((nb, m, f_out), jnp.float32),
        scratch_shapes=[pltpu.VMEM((n, nb * f_out), jnp.float32)],
        compiler_params=pltpu.CompilerParams(
            dimension_semantics=("arbitrary",),
        ),
    )(batch_input, weight, adj, bias2d)

    return out
